# baseline (speedup 1.0000x reference)
import functools

import jax
import jax.numpy as jnp
from jax import lax
from jax.experimental import pallas as pl
from jax.experimental.pallas import tpu as pltpu
from jax.experimental.pallas import tpu_sc as plsc

N_NODES = 10000
N_EDGES = 320000
D_EDGE = 16
D_NODE = 128
D_GLOBAL = 64
N_GRAPHS = 16

NC = 2
NS = 16
SUB = 128
IDX_ROWS = N_EDGES // SUB
ROWS_PER_TILE = IDX_ROWS // (NC * NS)
ROWS_PER_CHUNK = 6
NCHUNKS = ROWS_PER_TILE // ROWS_PER_CHUNK
CHUNK = SUB * ROWS_PER_CHUNK

NODE_BLK = 1000
N_BLKS = N_NODES // NODE_BLK


def _sc_scatter_body(ea_hbm, src_hbm, out_s_hbm, out_c_hbm,
                     vals_v, idx_v, ones_v, acc_sh, cnt_sh):
    c = lax.axis_index("c")
    s = lax.axis_index("s")
    w = s * NC + c

    zero16 = jnp.zeros((16,), jnp.float32)
    one16 = jnp.ones((16,), jnp.float32)

    def zrow(i, carry):
        vals_v[i, :] = zero16
        return carry

    lax.fori_loop(0, max(CHUNK, NODE_BLK), zrow, 0)

    def zrow1(i, carry):
        ones_v[pl.ds(i * 16, 16)] = zero16
        return carry

    n16 = max(CHUNK, NODE_BLK) // 16
    lax.fori_loop(0, n16, zrow1, 0)

    @pl.when(s < N_BLKS)
    def _zero_shared():
        pltpu.sync_copy(vals_v.at[pl.ds(0, NODE_BLK)],
                        acc_sh.at[pl.ds(s * NODE_BLK, NODE_BLK)])
        pltpu.sync_copy(ones_v.at[pl.ds(0, NODE_BLK)],
                        cnt_sh.at[pl.ds(s * NODE_BLK, NODE_BLK)])

    def orow(i, carry):
        ones_v[pl.ds(i * 16, 16)] = one16
        return carry

    lax.fori_loop(0, n16, orow, 0)

    plsc.subcore_barrier()

    def chunk_body(k, carry):
        row0 = w * ROWS_PER_TILE + k * ROWS_PER_CHUNK
        base = row0 * SUB
        pltpu.sync_copy(ea_hbm.at[pl.ds(base, CHUNK)], vals_v.at[pl.ds(0, CHUNK)])
        for j in range(ROWS_PER_CHUNK):
            pltpu.sync_copy(src_hbm.at[0, pl.ds(base + j * SUB, SUB)],
                            idx_v.at[j])
        for j in range(ROWS_PER_CHUNK):
            pltpu.sync_copy(vals_v.at[pl.ds(j * SUB, SUB)],
                            acc_sh.at[idx_v.at[j]], add=True)
            pltpu.sync_copy(ones_v.at[pl.ds(j * SUB, SUB)],
                            cnt_sh.at[idx_v.at[j]], add=True)
        return carry

    lax.fori_loop(0, NCHUNKS, chunk_body, 0)

    @pl.when(w < IDX_ROWS - ROWS_PER_TILE * NC * NS)
    def _tail():
        row = ROWS_PER_TILE * NC * NS + w
        pltpu.sync_copy(ea_hbm.at[pl.ds(row * SUB, SUB)],
                        vals_v.at[pl.ds(0, SUB)])
        pltpu.sync_copy(src_hbm.at[0, pl.ds(row * SUB, SUB)], idx_v.at[0])
        pltpu.sync_copy(vals_v.at[pl.ds(0, SUB)],
                        acc_sh.at[idx_v.at[0]], add=True)
        pltpu.sync_copy(ones_v.at[pl.ds(0, SUB)],
                        cnt_sh.at[idx_v.at[0]], add=True)

    plsc.subcore_barrier()

    @pl.when(s < N_BLKS)
    def _writeback():
        pltpu.sync_copy(acc_sh.at[pl.ds(s * NODE_BLK, NODE_BLK)],
                        vals_v.at[pl.ds(0, NODE_BLK)])
        pltpu.sync_copy(vals_v.at[pl.ds(0, NODE_BLK)],
                        out_s_hbm.at[c, pl.ds(s * NODE_BLK, NODE_BLK)])
        pltpu.sync_copy(cnt_sh.at[pl.ds(s * NODE_BLK, NODE_BLK)],
                        ones_v.at[pl.ds(0, NODE_BLK)])
        pltpu.sync_copy(ones_v.at[pl.ds(0, NODE_BLK)],
                        out_c_hbm.at[c, pl.ds(s * NODE_BLK, NODE_BLK)])


@functools.lru_cache(maxsize=1)
def _sc_scatter():
    return pl.kernel(
        _sc_scatter_body,
        out_type=(jax.ShapeDtypeStruct((NC, N_NODES, D_EDGE), jnp.float32),
                  jax.ShapeDtypeStruct((NC, N_NODES), jnp.float32)),
        mesh=plsc.VectorSubcoreMesh(core_axis_name="c", subcore_axis_name="s",
                                    num_cores=NC, num_subcores=NS),
        compiler_params=pltpu.CompilerParams(use_tc_tiling_on_sc=False),
        scratch_types=[
            pltpu.VMEM((max(CHUNK, NODE_BLK), D_EDGE), jnp.float32),
            pltpu.VMEM((ROWS_PER_CHUNK, SUB), jnp.int32),
            pltpu.VMEM((max(CHUNK, NODE_BLK),), jnp.float32),
            pltpu.VMEM_SHARED((N_NODES, D_EDGE), jnp.float32),
            pltpu.VMEM_SHARED((N_NODES,), jnp.float32),
        ],
    )


def _tc_mlp_body(x_ref, p_ref, c_ref, b_ref, u_ref, w1x_ref, w1e_ref,
                 w1u_ref, b1_ref, w2_ref, b2_ref, out_ref):
    f32 = jnp.float32
    p = p_ref[...]
    sums = p[0] + p[1]
    cnt = c_ref[...]
    denom = jnp.maximum(cnt[:, 0:1] + cnt[:, 1:2], 1.0)
    agg = sums / denom

    bvec = b_ref[0]
    ids = lax.broadcasted_iota(jnp.int32, (N_GRAPHS, NODE_BLK), 0)
    oht = (ids == jnp.broadcast_to(bvec, (N_GRAPHS, NODE_BLK))).astype(f32)

    uw = jnp.dot(u_ref[...], w1u_ref[...], preferred_element_type=f32)
    u_contrib = lax.dot_general(oht, uw, (((0,), (0,)), ((), ())),
                                preferred_element_type=f32)

    pre = (jnp.dot(x_ref[...], w1x_ref[...], preferred_element_type=f32)
           + jnp.dot(agg, w1e_ref[...], preferred_element_type=f32)
           + u_contrib + b1_ref[...])
    h = jnp.maximum(pre, 0.0)
    out_ref[...] = jnp.dot(h, w2_ref[...], preferred_element_type=f32) + b2_ref[...]


def kernel(x, edge_index, edge_attr, u, batch, W1, b1, W2, b2):
    f32 = jnp.float32
    sums2, cnt2 = _sc_scatter()(edge_attr.astype(f32),
                                edge_index.astype(jnp.int32))
    cnt_t = cnt2.T

    batch3 = batch.astype(jnp.int32).reshape(N_BLKS, 1, NODE_BLK)
    W1x = W1[:D_NODE]
    W1e = W1[D_NODE:D_NODE + D_EDGE]
    W1u = W1[D_NODE + D_EDGE:]
    b1r = b1.reshape(1, -1)
    b2r = b2.reshape(1, -1)

    out = pl.pallas_call(
        _tc_mlp_body,
        grid=(N_BLKS,),
        in_specs=[
            pl.BlockSpec((NODE_BLK, D_NODE), lambda i: (i, 0)),
            pl.BlockSpec((NC, NODE_BLK, D_EDGE), lambda i: (0, i, 0)),
            pl.BlockSpec((NODE_BLK, NC), lambda i: (i, 0)),
            pl.BlockSpec((1, 1, NODE_BLK), lambda i: (i, 0, 0)),
            pl.BlockSpec((N_GRAPHS, D_GLOBAL), lambda i: (0, 0)),
            pl.BlockSpec((D_NODE, 128), lambda i: (0, 0)),
            pl.BlockSpec((D_EDGE, 128), lambda i: (0, 0)),
            pl.BlockSpec((D_GLOBAL, 128), lambda i: (0, 0)),
            pl.BlockSpec((1, 128), lambda i: (0, 0)),
            pl.BlockSpec((128, 128), lambda i: (0, 0)),
            pl.BlockSpec((1, 128), lambda i: (0, 0)),
        ],
        out_specs=pl.BlockSpec((NODE_BLK, 128), lambda i: (i, 0)),
        out_shape=jax.ShapeDtypeStruct((N_NODES, 128), f32),
    )(x, sums2, cnt_t, batch3, u, W1x, W1e, W1u, b1r, W2, b2r)
    return out


# speedup vs baseline: 1.2589x; 1.2589x over previous
import functools

import jax
import jax.numpy as jnp
from jax import lax
from jax.experimental import pallas as pl
from jax.experimental.pallas import tpu as pltpu
from jax.experimental.pallas import tpu_sc as plsc

N_NODES = 10000
N_EDGES = 320000
D_EDGE = 16
D_NODE = 128
D_GLOBAL = 64
N_GRAPHS = 16

NC = 2
NS = 16
NW = NC * NS
SUB = 128
IDX_ROWS = N_EDGES // SUB
ROWS_PER_TILE = IDX_ROWS // NW
ROWS_PER_CHUNK = 6
NCHUNKS = ROWS_PER_TILE // ROWS_PER_CHUNK
CHUNK = SUB * ROWS_PER_CHUNK
N_TAIL = IDX_ROWS - ROWS_PER_TILE * NW

NODE_BLK = 1000
N_BLKS = N_NODES // NODE_BLK


def _fill(ref, n16, vec):
    def body(i, carry):
        ref[pl.ds(i * 16, 16)] = vec
        return carry

    lax.fori_loop(0, n16, body, 0)


def _sc_counts_body(ei_hbm, out_c_hbm, idx_v, ones_v, stage_v, cnt_sh, sem):
    c = lax.axis_index("c")
    s = lax.axis_index("s")
    w = s * NC + c

    zero16 = jnp.zeros((16,), jnp.float32)
    one16 = jnp.ones((16,), jnp.float32)
    _fill(stage_v, NODE_BLK // 16, zero16)
    _fill(ones_v, CHUNK // 16, one16)

    @pl.when(s < N_BLKS)
    def _zero_shared():
        pltpu.sync_copy(stage_v.at[pl.ds(0, NODE_BLK)],
                        cnt_sh.at[pl.ds(s * NODE_BLK, NODE_BLK)])

    plsc.subcore_barrier()

    def chunk_body(k, carry):
        row0 = w * ROWS_PER_TILE + k * ROWS_PER_CHUNK
        base = row0 * SUB
        loads = [pltpu.async_copy(ei_hbm.at[0, pl.ds(base + j * SUB, SUB)],
                                  idx_v.at[j], sem)
                 for j in range(ROWS_PER_CHUNK)]
        for d in loads:
            d.wait()
        scats = [pltpu.async_copy(ones_v.at[pl.ds(j * SUB, SUB)],
                                  cnt_sh.at[idx_v.at[j]], sem, add=True)
                 for j in range(ROWS_PER_CHUNK)]
        for d in scats:
            d.wait()
        return carry

    lax.fori_loop(0, NCHUNKS, chunk_body, 0)

    @pl.when(w < N_TAIL)
    def _tail():
        row = ROWS_PER_TILE * NW + w
        pltpu.sync_copy(ei_hbm.at[0, pl.ds(row * SUB, SUB)], idx_v.at[0])
        pltpu.sync_copy(ones_v.at[pl.ds(0, SUB)],
                        cnt_sh.at[idx_v.at[0]], add=True)

    plsc.subcore_barrier()

    @pl.when(s < N_BLKS)
    def _writeback():
        pltpu.sync_copy(cnt_sh.at[pl.ds(s * NODE_BLK, NODE_BLK)],
                        stage_v.at[pl.ds(0, NODE_BLK)])
        pltpu.sync_copy(stage_v.at[pl.ds(0, NODE_BLK)],
                        out_c_hbm.at[c, pl.ds(s * NODE_BLK, NODE_BLK)])


def _sc_values_body(ea_hbm, ei_hbm, out_s_hbm, vals_v, idx_v, acc_sh, sem):
    c = lax.axis_index("c")
    s = lax.axis_index("s")
    w = s * NC + c

    zero16 = jnp.zeros((16,), jnp.float32)

    def zrow(i, carry):
        vals_v[i, :] = zero16
        return carry

    lax.fori_loop(0, NODE_BLK, zrow, 0)

    @pl.when(s < N_BLKS)
    def _zero_shared():
        pltpu.sync_copy(vals_v.at[pl.ds(0, NODE_BLK)],
                        acc_sh.at[pl.ds(s * NODE_BLK, NODE_BLK)])

    plsc.subcore_barrier()

    def chunk_body(k, carry):
        row0 = w * ROWS_PER_TILE + k * ROWS_PER_CHUNK
        base = row0 * SUB
        loads = [pltpu.async_copy(ei_hbm.at[0, pl.ds(base + j * SUB, SUB)],
                                  idx_v.at[j], sem)
                 for j in range(ROWS_PER_CHUNK)]
        loads.append(pltpu.async_copy(ea_hbm.at[pl.ds(base, CHUNK)],
                                      vals_v.at[pl.ds(0, CHUNK)], sem))
        for d in loads:
            d.wait()
        scats = [pltpu.async_copy(vals_v.at[pl.ds(j * SUB, SUB)],
                                  acc_sh.at[idx_v.at[j]], sem, add=True)
                 for j in range(ROWS_PER_CHUNK)]
        for d in scats:
            d.wait()
        return carry

    lax.fori_loop(0, NCHUNKS, chunk_body, 0)

    @pl.when(w < N_TAIL)
    def _tail():
        row = ROWS_PER_TILE * NW + w
        pltpu.sync_copy(ea_hbm.at[pl.ds(row * SUB, SUB)],
                        vals_v.at[pl.ds(0, SUB)])
        pltpu.sync_copy(ei_hbm.at[0, pl.ds(row * SUB, SUB)], idx_v.at[0])
        pltpu.sync_copy(vals_v.at[pl.ds(0, SUB)],
                        acc_sh.at[idx_v.at[0]], add=True)

    plsc.subcore_barrier()

    @pl.when(s < N_BLKS)
    def _writeback():
        pltpu.sync_copy(acc_sh.at[pl.ds(s * NODE_BLK, NODE_BLK)],
                        vals_v.at[pl.ds(0, NODE_BLK)])
        pltpu.sync_copy(vals_v.at[pl.ds(0, NODE_BLK)],
                        out_s_hbm.at[c, pl.ds(s * NODE_BLK, NODE_BLK)])


@functools.lru_cache(maxsize=1)
def _sc_kernels():
    mesh = plsc.VectorSubcoreMesh(core_axis_name="c", subcore_axis_name="s",
                                  num_cores=NC, num_subcores=NS)
    params = pltpu.CompilerParams(use_tc_tiling_on_sc=False)
    counts = pl.kernel(
        _sc_counts_body,
        out_type=jax.ShapeDtypeStruct((NC, N_NODES), jnp.float32),
        mesh=mesh,
        compiler_params=params,
        scratch_types=[
            pltpu.VMEM((ROWS_PER_CHUNK, SUB), jnp.int32),
            pltpu.VMEM((CHUNK,), jnp.float32),
            pltpu.VMEM((NODE_BLK,), jnp.float32),
            pltpu.VMEM_SHARED((N_NODES,), jnp.float32),
            pltpu.SemaphoreType.DMA,
        ],
    )
    values = pl.kernel(
        _sc_values_body,
        out_type=jax.ShapeDtypeStruct((NC, N_NODES, D_EDGE), jnp.float32),
        mesh=mesh,
        compiler_params=params,
        scratch_types=[
            pltpu.VMEM((NODE_BLK, D_EDGE), jnp.float32),
            pltpu.VMEM((ROWS_PER_CHUNK, SUB), jnp.int32),
            pltpu.VMEM_SHARED((N_NODES, D_EDGE), jnp.float32),
            pltpu.SemaphoreType.DMA,
        ],
    )
    return counts, values


def _tc_mlp_body(x_ref, p_ref, c_ref, b_ref, u_ref, w1x_ref, w1e_ref,
                 w1u_ref, b1_ref, w2_ref, b2_ref, out_ref):
    f32 = jnp.float32
    p = p_ref[...]
    sums = p[0] + p[1]
    cnt = c_ref[...]
    denom = jnp.maximum(cnt[:, 0:1] + cnt[:, 1:2], 1.0)
    agg = sums / denom

    bvec = b_ref[0]
    ids = lax.broadcasted_iota(jnp.int32, (N_GRAPHS, NODE_BLK), 0)
    oht = (ids == jnp.broadcast_to(bvec, (N_GRAPHS, NODE_BLK))).astype(f32)

    uw = jnp.dot(u_ref[...], w1u_ref[...], preferred_element_type=f32)
    u_contrib = lax.dot_general(oht, uw, (((0,), (0,)), ((), ())),
                                preferred_element_type=f32)

    pre = (jnp.dot(x_ref[...], w1x_ref[...], preferred_element_type=f32)
           + jnp.dot(agg, w1e_ref[...], preferred_element_type=f32)
           + u_contrib + b1_ref[...])
    h = jnp.maximum(pre, 0.0)
    out_ref[...] = jnp.dot(h, w2_ref[...], preferred_element_type=f32) + b2_ref[...]


def kernel(x, edge_index, edge_attr, u, batch, W1, b1, W2, b2):
    f32 = jnp.float32
    ei32 = edge_index.astype(jnp.int32)
    counts_k, values_k = _sc_kernels()
    cnt2 = counts_k(ei32)
    sums2 = values_k(edge_attr.astype(f32), ei32)
    cnt_t = cnt2.T

    batch3 = batch.astype(jnp.int32).reshape(N_BLKS, 1, NODE_BLK)
    W1x = W1[:D_NODE]
    W1e = W1[D_NODE:D_NODE + D_EDGE]
    W1u = W1[D_NODE + D_EDGE:]
    b1r = b1.reshape(1, -1)
    b2r = b2.reshape(1, -1)

    out = pl.pallas_call(
        _tc_mlp_body,
        grid=(N_BLKS,),
        in_specs=[
            pl.BlockSpec((NODE_BLK, D_NODE), lambda i: (i, 0)),
            pl.BlockSpec((NC, NODE_BLK, D_EDGE), lambda i: (0, i, 0)),
            pl.BlockSpec((NODE_BLK, NC), lambda i: (i, 0)),
            pl.BlockSpec((1, 1, NODE_BLK), lambda i: (i, 0, 0)),
            pl.BlockSpec((N_GRAPHS, D_GLOBAL), lambda i: (0, 0)),
            pl.BlockSpec((D_NODE, 128), lambda i: (0, 0)),
            pl.BlockSpec((D_EDGE, 128), lambda i: (0, 0)),
            pl.BlockSpec((D_GLOBAL, 128), lambda i: (0, 0)),
            pl.BlockSpec((1, 128), lambda i: (0, 0)),
            pl.BlockSpec((128, 128), lambda i: (0, 0)),
            pl.BlockSpec((1, 128), lambda i: (0, 0)),
        ],
        out_specs=pl.BlockSpec((NODE_BLK, 128), lambda i: (i, 0)),
        out_shape=jax.ShapeDtypeStruct((N_NODES, 128), f32),
    )(x, sums2, cnt_t, batch3, u, W1x, W1e, W1u, b1r, W2, b2r)
    return out
